# Initial kernel scaffold; baseline (speedup 1.0000x reference)
import jax
import jax.numpy as jnp
from jax.experimental import pallas as pl


def kernel(atom_features, r, edge_index, W_emb, b_emb, W_src, b_src, W_dst, b_dst, W_edge, b_edge, g_msg, be_msg, g_bn, be_bn, W_fc, b_fc, W_out, b_out):
    raise NotImplementedError("write your pallas kernel here")


# speedup vs baseline: 1.9131x; 1.9131x over previous
import functools

import jax
import jax.numpy as jnp
from jax import lax
from jax.experimental import pallas as pl
from jax.experimental.pallas import tpu as pltpu
from jax.experimental.pallas import tpu_sc as plsc

N = 10000
E = 320000
D_IN = 92
D = 64
D2 = 128
D_EDGE = 41
FC = 128
L = 3
EPS = 1e-5

NC = 2
NS = 16
NW = NC * NS
CHUNK = 128
NCHUNK = E // CHUNK
SC_ITERS = (NCHUNK + NW - 1) // NW
ZROWS = 624
ZLAST = N - ZROWS * (NS - 1)

TN = 2000
NT = N // TN
TE = 2000
ET = E // TE

_RBF_STEP = 8.0 / (D_EDGE - 1)
_RBF_GAMMA = 1.0 / (_RBF_STEP * _RBF_STEP)


def _sc_gather_body(hsrc, hdst, src2d, dst2d, g_out, idxa, idxb, rows, sem):
    c = lax.axis_index("c")
    s = lax.axis_index("s")
    w = s * NC + c

    def body(i, carry):
        chunk = w + NW * i

        @pl.when(chunk < NCHUNK)
        def _():
            pltpu.sync_copy(src2d.at[chunk], idxa)
            pltpu.sync_copy(dst2d.at[chunk], idxb)
            pltpu.async_copy(hsrc.at[idxa], rows, sem).wait()
            pltpu.async_copy(hdst.at[idxb], rows, sem, add=True).wait()
            pltpu.sync_copy(rows, g_out.at[pl.ds(chunk * CHUNK, CHUNK), :])

        return carry

    lax.fori_loop(0, SC_ITERS, body, 0)


@functools.lru_cache(maxsize=None)
def _sc_gather():
    return pl.kernel(
        _sc_gather_body,
        out_type=jax.ShapeDtypeStruct((E, D2), jnp.float32),
        mesh=plsc.VectorSubcoreMesh(core_axis_name="c", subcore_axis_name="s"),
        scratch_types=[
            pltpu.VMEM((CHUNK,), jnp.int32),
            pltpu.VMEM((CHUNK,), jnp.int32),
            pltpu.VMEM((CHUNK, D2), jnp.float32),
            pltpu.SemaphoreType.DMA,
        ],
    )


def _sc_scatter_body(mg, dst2d, out_flat, idxw, rows, zrow, acc, sem):
    c = lax.axis_index("c")
    s = lax.axis_index("s")
    w = s * NC + c

    def zbody(i, carry):
        zrow[i // 4, pl.ds((i % 4) * 16, 16)] = jnp.zeros((16,), jnp.float32)
        return carry

    lax.fori_loop(0, 16 * 4, zbody, 0)
    off = s * ZROWS
    nz = jnp.where(s == NS - 1, ZLAST // 16, ZROWS // 16)

    def zcopy(j, carry):
        pltpu.sync_copy(zrow, acc.at[pl.ds(off + j * 16, 16), :])
        return carry

    lax.fori_loop(0, nz, zcopy, 0)
    plsc.subcore_barrier()

    def body(i, carry):
        chunk = w + NW * i

        @pl.when(chunk < NCHUNK)
        def _():
            pltpu.sync_copy(dst2d.at[chunk], idxw)
            pltpu.sync_copy(mg.at[pl.ds(chunk * CHUNK, CHUNK), :], rows)
            pltpu.async_copy(rows, acc.at[idxw], sem, add=True).wait()

        return carry

    lax.fori_loop(0, SC_ITERS, body, 0)
    plsc.subcore_barrier()

    @pl.when(s < NS - 1)
    def _():
        pltpu.sync_copy(
            acc.at[pl.ds(off, ZROWS), :],
            out_flat.at[pl.ds(c * N + off, ZROWS), :],
        )

    @pl.when(s == NS - 1)
    def _():
        pltpu.sync_copy(
            acc.at[pl.ds(off, ZLAST), :],
            out_flat.at[pl.ds(c * N + off, ZLAST), :],
        )


@functools.lru_cache(maxsize=None)
def _sc_scatter():
    return pl.kernel(
        _sc_scatter_body,
        out_type=jax.ShapeDtypeStruct((2 * N, D), jnp.float32),
        mesh=plsc.VectorSubcoreMesh(core_axis_name="c", subcore_axis_name="s"),
        scratch_types=[
            pltpu.VMEM((CHUNK,), jnp.int32),
            pltpu.VMEM((CHUNK, D), jnp.float32),
            pltpu.VMEM((16, D), jnp.float32),
            pltpu.VMEM_SHARED((N, D), jnp.float32),
            pltpu.SemaphoreType.DMA,
        ],
    )


def _embed_body(x_ref, w_ref, b_ref, o_ref):
    o_ref[...] = (
        jnp.dot(x_ref[...], w_ref[...], preferred_element_type=jnp.float32)
        + b_ref[...]
    )


@functools.lru_cache(maxsize=None)
def _embed():
    return pl.pallas_call(
        _embed_body,
        grid=(NT,),
        in_specs=[
            pl.BlockSpec((TN, D_IN), lambda i: (i, 0)),
            pl.BlockSpec((D_IN, D), lambda i: (0, 0)),
            pl.BlockSpec((1, D), lambda i: (0, 0)),
        ],
        out_specs=pl.BlockSpec((TN, D), lambda i: (i, 0)),
        out_shape=jax.ShapeDtypeStruct((N, D), jnp.float32),
    )


def _dense_body(h_ref, w_ref, b_ref, s_ref, d_ref):
    hb = jnp.dot(h_ref[...], w_ref[...], preferred_element_type=jnp.float32)
    hb = hb + b_ref[...]
    s_ref[...] = hb[:, :D2]
    d_ref[...] = hb[:, D2:]


@functools.lru_cache(maxsize=None)
def _dense():
    return pl.pallas_call(
        _dense_body,
        grid=(NT,),
        in_specs=[
            pl.BlockSpec((TN, D), lambda i: (i, 0)),
            pl.BlockSpec((D, 2 * D2), lambda i: (0, 0)),
            pl.BlockSpec((1, 2 * D2), lambda i: (0, 0)),
        ],
        out_specs=[
            pl.BlockSpec((TN, D2), lambda i: (i, 0)),
            pl.BlockSpec((TN, D2), lambda i: (i, 0)),
        ],
        out_shape=[
            jax.ShapeDtypeStruct((N, D2), jnp.float32),
            jax.ShapeDtypeStruct((N, D2), jnp.float32),
        ],
    )


def _edge_m(g_ref, r_ref, wpad_ref, bed_ref):
    rr = r_ref[...]
    bl = jnp.sqrt(jnp.sum(rr * rr, axis=1, keepdims=True))
    k = lax.broadcasted_iota(jnp.int32, (1, D2), 1).astype(jnp.float32)
    phi = jnp.exp(-_RBF_GAMMA * (bl - k * _RBF_STEP) ** 2)
    eproj = (
        jnp.dot(phi, wpad_ref[...], preferred_element_type=jnp.float32)
        + bed_ref[...]
    )
    return g_ref[...] + eproj


def _edge_body(g_ref, r_ref, wpad_ref, bed_ref, gm_ref, bm_ref, mg_ref,
               accs, accq, svec, tvec):
    p = pl.program_id(0)
    e = pl.program_id(1)

    @pl.when(jnp.logical_and(p == 0, e == 0))
    def _():
        accs[...] = jnp.zeros_like(accs)
        accq[...] = jnp.zeros_like(accq)

    m = _edge_m(g_ref, r_ref, wpad_ref, bed_ref)

    @pl.when(p == 0)
    def _():
        accs[...] += jnp.sum(m, axis=0, keepdims=True)
        accq[...] += jnp.sum(m * m, axis=0, keepdims=True)

    @pl.when(jnp.logical_and(p == 0, e == ET - 1))
    def _():
        mean = accs[...] * (1.0 / E)
        var = accq[...] * (1.0 / E) - mean * mean
        rstd = lax.rsqrt(var + EPS)
        svec[...] = gm_ref[...] * rstd
        tvec[...] = bm_ref[...] - mean * gm_ref[...] * rstd

    @pl.when(p == 1)
    def _():
        mn = m * svec[...] + tvec[...]
        mg_ref[...] = jax.nn.sigmoid(mn[:, :D]) * jax.nn.softplus(mn[:, D:])


@functools.lru_cache(maxsize=None)
def _edge():
    return pl.pallas_call(
        _edge_body,
        grid=(2, ET),
        in_specs=[
            pl.BlockSpec((TE, D2), lambda p, e: (e, 0)),
            pl.BlockSpec((TE, 3), lambda p, e: (e, 0)),
            pl.BlockSpec((D2, D2), lambda p, e: (0, 0)),
            pl.BlockSpec((1, D2), lambda p, e: (0, 0)),
            pl.BlockSpec((1, D2), lambda p, e: (0, 0)),
            pl.BlockSpec((1, D2), lambda p, e: (0, 0)),
        ],
        out_specs=pl.BlockSpec((TE, D), lambda p, e: (e, 0)),
        out_shape=jax.ShapeDtypeStruct((E, D), jnp.float32),
        scratch_shapes=[
            pltpu.VMEM((1, D2), jnp.float32),
            pltpu.VMEM((1, D2), jnp.float32),
            pltpu.VMEM((1, D2), jnp.float32),
            pltpu.VMEM((1, D2), jnp.float32),
        ],
    )


def _update_body(a0_ref, a1_ref, h_ref, gb_ref, bb_ref, h_out, hsum_out,
                 accs, accq, svec, tvec, hacc):
    p = pl.program_id(0)
    n = pl.program_id(1)

    @pl.when(jnp.logical_and(p == 0, n == 0))
    def _():
        accs[...] = jnp.zeros_like(accs)
        accq[...] = jnp.zeros_like(accq)
        hacc[...] = jnp.zeros_like(hacc)

    a = a0_ref[...] + a1_ref[...]

    @pl.when(p == 0)
    def _():
        accs[...] += jnp.sum(a, axis=0, keepdims=True)
        accq[...] += jnp.sum(a * a, axis=0, keepdims=True)

    @pl.when(jnp.logical_and(p == 0, n == NT - 1))
    def _():
        mean = accs[...] * (1.0 / N)
        var = accq[...] * (1.0 / N) - mean * mean
        rstd = lax.rsqrt(var + EPS)
        svec[...] = gb_ref[...] * rstd
        tvec[...] = bb_ref[...] - mean * gb_ref[...] * rstd

    @pl.when(p == 1)
    def _():
        hn = jax.nn.softplus(h_ref[...] + a * svec[...] + tvec[...])
        h_out[...] = hn
        hacc[...] += jnp.sum(hn, axis=0, keepdims=True)
        hsum_out[...] = hacc[...]


@functools.lru_cache(maxsize=None)
def _update():
    return pl.pallas_call(
        _update_body,
        grid=(2, NT),
        in_specs=[
            pl.BlockSpec((TN, D), lambda p, n: (n, 0)),
            pl.BlockSpec((TN, D), lambda p, n: (NT + n, 0)),
            pl.BlockSpec((TN, D), lambda p, n: (n, 0)),
            pl.BlockSpec((1, D), lambda p, n: (0, 0)),
            pl.BlockSpec((1, D), lambda p, n: (0, 0)),
        ],
        out_specs=[
            pl.BlockSpec((TN, D), lambda p, n: (n, 0)),
            pl.BlockSpec((1, D), lambda p, n: (0, 0)),
        ],
        out_shape=[
            jax.ShapeDtypeStruct((N, D), jnp.float32),
            jax.ShapeDtypeStruct((1, D), jnp.float32),
        ],
        scratch_shapes=[
            pltpu.VMEM((1, D), jnp.float32),
            pltpu.VMEM((1, D), jnp.float32),
            pltpu.VMEM((1, D), jnp.float32),
            pltpu.VMEM((1, D), jnp.float32),
            pltpu.VMEM((1, D), jnp.float32),
        ],
    )


def _readout_body(hs_ref, wfc_ref, bfc_ref, woutt_ref, bout_ref, o_ref):
    f = jax.nn.softplus(hs_ref[...] * (1.0 / N))
    f = jax.nn.softplus(
        jnp.dot(f, wfc_ref[...], preferred_element_type=jnp.float32)
        + bfc_ref[...]
    )
    f = jax.nn.softplus(f)
    o_ref[...] = jnp.sum(f * woutt_ref[...], axis=1, keepdims=True) + bout_ref[...]


@functools.lru_cache(maxsize=None)
def _readout():
    return pl.pallas_call(
        _readout_body,
        in_specs=[
            pl.BlockSpec((1, D), lambda: (0, 0)),
            pl.BlockSpec((D, FC), lambda: (0, 0)),
            pl.BlockSpec((1, FC), lambda: (0, 0)),
            pl.BlockSpec((1, FC), lambda: (0, 0)),
            pl.BlockSpec((1, 1), lambda: (0, 0)),
        ],
        out_specs=pl.BlockSpec((1, 1), lambda: (0, 0)),
        out_shape=jax.ShapeDtypeStruct((1, 1), jnp.float32),
    )


def kernel(atom_features, r, edge_index, W_emb, b_emb, W_src, b_src, W_dst,
           b_dst, W_edge, b_edge, g_msg, be_msg, g_bn, be_bn, W_fc, b_fc,
           W_out, b_out):
    p = jnp.argsort(edge_index[1])
    p = p.reshape(CHUNK, NCHUNK).T.reshape(E)
    src_p = edge_index[0][p]
    dst_p = edge_index[1][p]
    r_p = r[p]
    src2d = src_p.reshape(NCHUNK, CHUNK)
    dst2d = dst_p.reshape(NCHUNK, CHUNK)

    h = _embed()(atom_features, W_emb, b_emb.reshape(1, D))
    hsum = None
    for l in range(L):
        wcat = jnp.concatenate([W_src[l], W_dst[l]], axis=1)
        bcat = jnp.concatenate([b_src[l], b_dst[l]]).reshape(1, 2 * D2)
        hsrc, hdst = _dense()(h, wcat, bcat)
        g = _sc_gather()(hsrc, hdst, src2d, dst2d)
        wpad = jnp.pad(W_edge[l], ((0, D2 - D_EDGE), (0, 0)))
        mg = _edge()(g, r_p, wpad, b_edge[l].reshape(1, D2),
                     g_msg[l].reshape(1, D2), be_msg[l].reshape(1, D2))
        agg2 = _sc_scatter()(mg, dst2d)
        h, hsum = _update()(agg2, agg2, h, g_bn[l].reshape(1, D),
                            be_bn[l].reshape(1, D))
    out = _readout()(hsum, W_fc, b_fc.reshape(1, FC), W_out.reshape(1, FC),
                     b_out.reshape(1, 1))
    return out[0, 0]
